# Initial kernel scaffold; baseline (speedup 1.0000x reference)
import jax
import jax.numpy as jnp
from jax.experimental import pallas as pl


def kernel(x, gate_w, fc1_w, fc2_w, proj_w):
    raise NotImplementedError("write your pallas kernel here")


# speedup vs baseline: 2.7394x; 2.7394x over previous
import functools

import jax
import jax.numpy as jnp
from jax.experimental import pallas as pl
from jax.experimental.pallas import tpu as pltpu

T_BLK = 512


def _moe_dense_kernel(x_ref, gate_ref, fc1_ref, fc2_ref, proj_ref, y_ref,
                      comb_ref):
    e = pl.program_id(1)
    n_e = pl.num_programs(1)

    @pl.when(e == 0)
    def _():
        xb = x_ref[...]
        router = jax.lax.dot_general(
            xb, gate_ref[...], (((1,), (1,)), ((), ())),
            preferred_element_type=jnp.float32)
        m1 = jnp.max(router, axis=1, keepdims=True)
        ids = jax.lax.broadcasted_iota(jnp.int32, router.shape, 1)
        i1 = jnp.min(jnp.where(router == m1, ids, n_e), axis=1, keepdims=True)
        masked = jnp.where(ids == i1, -jnp.inf, router)
        m2 = jnp.max(masked, axis=1, keepdims=True)
        i2 = jnp.min(jnp.where(masked == m2, ids, n_e), axis=1, keepdims=True)
        p1 = 1.0 / (1.0 + jnp.exp(m2 - m1))
        p2 = 1.0 - p1
        comb_ref[...] = jnp.where(ids == i1, p1, 0.0) + jnp.where(
            ids == i2, p2, 0.0)

    xb = x_ref[...]
    h = jax.lax.dot_general(xb, fc1_ref[0], (((1,), (1,)), ((), ())),
                            preferred_element_type=jnp.float32)
    g = jax.lax.dot_general(xb, fc2_ref[0], (((1,), (1,)), ((), ())),
                            preferred_element_type=jnp.float32)
    a = (h * jax.lax.logistic(h)) * g
    o = jax.lax.dot_general(a, proj_ref[0], (((1,), (1,)), ((), ())),
                            preferred_element_type=jnp.float32)
    ids = jax.lax.broadcasted_iota(jnp.int32, comb_ref.shape, 1)
    w = jnp.sum(jnp.where(ids == e, comb_ref[...], 0.0), axis=1,
                keepdims=True)
    contrib = w * o

    @pl.when(e == 0)
    def _():
        y_ref[...] = contrib

    @pl.when(e != 0)
    def _():
        y_ref[...] += contrib


@jax.jit
def kernel(x, gate_w, fc1_w, fc2_w, proj_w):
    B, T, C = x.shape
    E, F, _ = fc1_w.shape
    xf = x.reshape(T, C)
    grid = (T // T_BLK, E)
    y = pl.pallas_call(
        _moe_dense_kernel,
        grid=grid,
        in_specs=[
            pl.BlockSpec((T_BLK, C), lambda t, e: (t, 0)),
            pl.BlockSpec((E, C), lambda t, e: (0, 0)),
            pl.BlockSpec((1, F, C), lambda t, e: (e, 0, 0)),
            pl.BlockSpec((1, F, C), lambda t, e: (e, 0, 0)),
            pl.BlockSpec((1, C, F), lambda t, e: (e, 0, 0)),
        ],
        out_specs=pl.BlockSpec((T_BLK, C), lambda t, e: (t, 0)),
        out_shape=jax.ShapeDtypeStruct((T, C), jnp.float32),
        scratch_shapes=[pltpu.VMEM((T_BLK, E), jnp.float32)],
    )(xf, gate_w, fc1_w, fc2_w, proj_w)
    return y.reshape(B, T, C)


# speedup vs baseline: 3.6974x; 1.3497x over previous
import functools

import jax
import jax.numpy as jnp
from jax.experimental import pallas as pl
from jax.experimental.pallas import tpu as pltpu

T_BLK = 1024


def _moe_dense_kernel(x_ref, gate_ref, fc1_ref, fc2_ref, proj_ref, y_ref,
                      comb_ref):
    e = pl.program_id(1)
    n_e = pl.num_programs(1)

    @pl.when(e == 0)
    def _():
        xb = x_ref[...]
        router = jax.lax.dot_general(
            xb, gate_ref[...], (((1,), (1,)), ((), ())),
            preferred_element_type=jnp.float32)
        m1 = jnp.max(router, axis=1, keepdims=True)
        ids = jax.lax.broadcasted_iota(jnp.int32, router.shape, 1)
        i1 = jnp.min(jnp.where(router == m1, ids, n_e), axis=1, keepdims=True)
        masked = jnp.where(ids == i1, -jnp.inf, router)
        m2 = jnp.max(masked, axis=1, keepdims=True)
        i2 = jnp.min(jnp.where(masked == m2, ids, n_e), axis=1, keepdims=True)
        p1 = 1.0 / (1.0 + jnp.exp(m2 - m1))
        p2 = 1.0 - p1
        comb_ref[...] = jnp.where(ids == i1, p1, 0.0) + jnp.where(
            ids == i2, p2, 0.0)

    xb = x_ref[...]
    h = jax.lax.dot_general(xb, fc1_ref[0], (((1,), (1,)), ((), ())),
                            preferred_element_type=jnp.float32)
    g = jax.lax.dot_general(xb, fc2_ref[0], (((1,), (1,)), ((), ())),
                            preferred_element_type=jnp.float32)
    a = (h * jax.lax.logistic(h)) * g
    o = jax.lax.dot_general(a, proj_ref[0], (((1,), (1,)), ((), ())),
                            preferred_element_type=jnp.float32)
    ids = jax.lax.broadcasted_iota(jnp.int32, comb_ref.shape, 1)
    w = jnp.sum(jnp.where(ids == e, comb_ref[...], 0.0), axis=1,
                keepdims=True)
    contrib = w * o

    @pl.when(e == 0)
    def _():
        y_ref[...] = contrib

    @pl.when(e != 0)
    def _():
        y_ref[...] += contrib


@jax.jit
def kernel(x, gate_w, fc1_w, fc2_w, proj_w):
    B, T, C = x.shape
    E, F, _ = fc1_w.shape
    xf = x.reshape(T, C)
    grid = (T // T_BLK, E)
    y = pl.pallas_call(
        _moe_dense_kernel,
        grid=grid,
        in_specs=[
            pl.BlockSpec((T_BLK, C), lambda t, e: (t, 0)),
            pl.BlockSpec((E, C), lambda t, e: (0, 0)),
            pl.BlockSpec((1, F, C), lambda t, e: (e, 0, 0)),
            pl.BlockSpec((1, F, C), lambda t, e: (e, 0, 0)),
            pl.BlockSpec((1, C, F), lambda t, e: (e, 0, 0)),
        ],
        out_specs=pl.BlockSpec((T_BLK, C), lambda t, e: (t, 0)),
        out_shape=jax.ShapeDtypeStruct((T, C), jnp.float32),
        scratch_shapes=[pltpu.VMEM((T_BLK, E), jnp.float32)],
    )(xf, gate_w, fc1_w, fc2_w, proj_w)
    return y.reshape(B, T, C)


# speedup vs baseline: 3.8894x; 1.0519x over previous
import functools

import jax
import jax.numpy as jnp
from jax.experimental import pallas as pl
from jax.experimental.pallas import tpu as pltpu

T_BLK = 2048


def _moe_dense_kernel(x_ref, gate_ref, fc1_ref, fc2_ref, proj_ref, y_ref,
                      comb_ref):
    e = pl.program_id(1)
    n_e = pl.num_programs(1)

    @pl.when(e == 0)
    def _():
        xb = x_ref[...]
        router = jax.lax.dot_general(
            xb, gate_ref[...], (((1,), (1,)), ((), ())),
            preferred_element_type=jnp.float32)
        m1 = jnp.max(router, axis=1, keepdims=True)
        ids = jax.lax.broadcasted_iota(jnp.int32, router.shape, 1)
        i1 = jnp.min(jnp.where(router == m1, ids, n_e), axis=1, keepdims=True)
        masked = jnp.where(ids == i1, -jnp.inf, router)
        m2 = jnp.max(masked, axis=1, keepdims=True)
        i2 = jnp.min(jnp.where(masked == m2, ids, n_e), axis=1, keepdims=True)
        p1 = 1.0 / (1.0 + jnp.exp(m2 - m1))
        p2 = 1.0 - p1
        comb_ref[...] = jnp.where(ids == i1, p1, 0.0) + jnp.where(
            ids == i2, p2, 0.0)

    xb = x_ref[...]
    h = jax.lax.dot_general(xb, fc1_ref[0], (((1,), (1,)), ((), ())),
                            preferred_element_type=jnp.float32)
    g = jax.lax.dot_general(xb, fc2_ref[0], (((1,), (1,)), ((), ())),
                            preferred_element_type=jnp.float32)
    a = (h * jax.lax.logistic(h)) * g
    o = jax.lax.dot_general(a, proj_ref[0], (((1,), (1,)), ((), ())),
                            preferred_element_type=jnp.float32)
    ids = jax.lax.broadcasted_iota(jnp.int32, comb_ref.shape, 1)
    w = jnp.sum(jnp.where(ids == e, comb_ref[...], 0.0), axis=1,
                keepdims=True)
    contrib = w * o

    @pl.when(e == 0)
    def _():
        y_ref[...] = contrib

    @pl.when(e != 0)
    def _():
        y_ref[...] += contrib


@jax.jit
def kernel(x, gate_w, fc1_w, fc2_w, proj_w):
    B, T, C = x.shape
    E, F, _ = fc1_w.shape
    xf = x.reshape(T, C)
    grid = (T // T_BLK, E)
    y = pl.pallas_call(
        _moe_dense_kernel,
        grid=grid,
        in_specs=[
            pl.BlockSpec((T_BLK, C), lambda t, e: (t, 0)),
            pl.BlockSpec((E, C), lambda t, e: (0, 0)),
            pl.BlockSpec((1, F, C), lambda t, e: (e, 0, 0)),
            pl.BlockSpec((1, F, C), lambda t, e: (e, 0, 0)),
            pl.BlockSpec((1, C, F), lambda t, e: (e, 0, 0)),
        ],
        out_specs=pl.BlockSpec((T_BLK, C), lambda t, e: (t, 0)),
        out_shape=jax.ShapeDtypeStruct((T, C), jnp.float32),
        scratch_shapes=[pltpu.VMEM((T_BLK, E), jnp.float32)],
    )(xf, gate_w, fc1_w, fc2_w, proj_w)
    return y.reshape(B, T, C)
